# Initial kernel scaffold; baseline (speedup 1.0000x reference)
import jax
import jax.numpy as jnp
from jax.experimental import pallas as pl


def kernel(points, params):
    raise NotImplementedError("write your pallas kernel here")


# speedup vs baseline: 1.9477x; 1.9477x over previous
import functools

import jax
import jax.numpy as jnp
from jax import lax
from jax.experimental import pallas as pl
from jax.experimental.pallas import tpu as pltpu

_K = 20
_EPS = 1e-5
_T = 8


def _knn_conv_body(d_true, x_ref, w_ref, bias_ref,
                   up_ref, um_ref, s1_ref, s2_ref):
  t = pl.program_id(1)
  n = x_ref.shape[1]
  r = n // _T
  h = w_ref.shape[1]

  xb = x_ref[0]
  xt = x_ref[0, pl.ds(t * r, r), :]
  sq_rows = jnp.sum(xt * xt, axis=1, keepdims=True)
  ones_row = jnp.ones((1, xb.shape[1]), jnp.float32)
  sq_cols = lax.dot_general(ones_row, xb * xb,
                            (((1,), (1,)), ((), ())),
                            preferred_element_type=jnp.float32,
                            precision=lax.Precision.HIGHEST)
  cross = lax.dot_general(xt, xb, (((1,), (1,)), ((), ())),
                          preferred_element_type=jnp.float32,
                          precision=lax.Precision.DEFAULT)
  d2 = sq_rows + sq_cols - 2.0 * cross
  ji = lax.broadcasted_iota(jnp.int32, (r, n), 1)
  ri = lax.broadcasted_iota(jnp.int32, (r, n), 0)
  d2 = jnp.where(ji == ri + t * r, d2 + 1e10, d2)

  xb_d = xb[:, :d_true]
  xt_d = xt[:, :d_true]
  hmax = jnp.full((r, h), -jnp.inf, jnp.float32)
  hmin = jnp.full((r, h), jnp.inf, jnp.float32)
  hsum = jnp.zeros((r, h), jnp.float32)
  hsum_c = jnp.zeros((r, h), jnp.float32)
  hsq = jnp.zeros((r, h), jnp.float32)
  hsq_c = jnp.zeros((r, h), jnp.float32)
  for _ in range(_K):
    v = jnp.min(d2, axis=1, keepdims=True)
    idx = jnp.min(jnp.where(d2 == v, ji, n), axis=1, keepdims=True)
    sel = ji == idx
    xj = lax.dot_general(sel.astype(jnp.float32), xb_d,
                         (((1,), (0,)), ((), ())),
                         preferred_element_type=jnp.float32,
                         precision=lax.Precision.HIGHEST)
    feat = jnp.concatenate([xt_d, xj - xt_d], axis=1)
    hs = lax.dot_general(feat, w_ref[...], (((1,), (0,)), ((), ())),
                         preferred_element_type=jnp.float32,
                         precision=lax.Precision.DEFAULT) + bias_ref[...]
    hmax = jnp.maximum(hmax, hs)
    hmin = jnp.minimum(hmin, hs)
    y1 = hs - hsum_c
    t1 = hsum + y1
    hsum_c = (t1 - hsum) - y1
    hsum = t1
    hs2 = hs * hs
    y2 = hs2 - hsq_c
    t2 = hsq + y2
    hsq_c = (t2 - hsq) - y2
    hsq = t2
    d2 = jnp.where(sel, jnp.inf, d2)

  up_ref[0] = hmax
  um_ref[0] = hmin
  s1 = jnp.sum(hsum, axis=0, keepdims=True) - jnp.sum(hsum_c, axis=0,
                                                      keepdims=True)
  s2 = jnp.sum(hsq, axis=0, keepdims=True) - jnp.sum(hsq_c, axis=0,
                                                     keepdims=True)
  s1_ref[...] = s1.reshape(1, 1, 1, h)
  s2_ref[...] = s2.reshape(1, 1, 1, h)


def _knn_conv(x, w, bias, d_true):
  b, n, dp = x.shape
  h = w.shape[1]
  return pl.pallas_call(
      functools.partial(_knn_conv_body, d_true),
      grid=(b, _T),
      in_specs=[
          pl.BlockSpec((1, n, dp), lambda bb, tt: (bb, 0, 0)),
          pl.BlockSpec(w.shape, lambda bb, tt: (0, 0)),
          pl.BlockSpec((1, h), lambda bb, tt: (0, 0)),
      ],
      out_specs=[
          pl.BlockSpec((1, n // _T, h), lambda bb, tt: (bb, tt, 0)),
          pl.BlockSpec((1, n // _T, h), lambda bb, tt: (bb, tt, 0)),
          pl.BlockSpec((1, 1, 1, h), lambda bb, tt: (bb, tt, 0, 0)),
          pl.BlockSpec((1, 1, 1, h), lambda bb, tt: (bb, tt, 0, 0)),
      ],
      out_shape=[
          jax.ShapeDtypeStruct((b, n, h), jnp.float32),
          jax.ShapeDtypeStruct((b, n, h), jnp.float32),
          jax.ShapeDtypeStruct((b, _T, 1, h), jnp.float32),
          jax.ShapeDtypeStruct((b, _T, 1, h), jnp.float32),
      ],
  )(x, w, bias)


def _finalize_body(up_ref, um_ref, s1_ref, s2_ref,
                   cg_ref, cb_ref, g2_ref, b2_ref, xn_ref):
  b, n, h = up_ref.shape
  cnt = b * n * _K
  s1 = jnp.sum(s1_ref[...].reshape(b * _T, h), axis=0, keepdims=True)
  s2 = jnp.sum(s2_ref[...].reshape(b * _T, h), axis=0, keepdims=True)
  m = s1 / cnt
  var = s2 / cnt - m * m
  sd = jnp.sqrt(var + _EPS)
  cg = cg_ref[...]
  pick_max = jnp.broadcast_to((cg >= 0.0).reshape(1, 1, h), (b, n, h))
  u = jnp.where(pick_max, up_ref[...], um_ref[...]).reshape(b * n, h)
  y = jnp.maximum(cg * (u - m) / sd + cb_ref[...], 0.0)
  m2 = jnp.mean(y, axis=0, keepdims=True)
  dev = y - m2
  v2 = jnp.mean(dev * dev, axis=0, keepdims=True)
  xn = jnp.maximum(g2_ref[...] * dev / jnp.sqrt(v2 + _EPS) + b2_ref[...],
                   0.0)
  xn_ref[...] = xn.reshape(b, n, h)


def _finalize(up, um, s1p, s2p, cg, cb, g2, b2):
  b, n, h = up.shape
  return pl.pallas_call(
      _finalize_body,
      out_shape=jax.ShapeDtypeStruct((b, n, h), jnp.float32),
  )(up, um, s1p, s2p, cg, cb, g2, b2)


def _head_body(x_ref, w1_ref, b1_ref, g1_ref, bb1_ref,
               w2_ref, b2_ref, g2_ref, bb2_ref, out_ref):
  n = x_ref.shape[1]
  pooled = jnp.sum(x_ref[...], axis=1) / n
  h1 = jnp.dot(pooled, w1_ref[...],
               preferred_element_type=jnp.float32,
               precision=lax.Precision.DEFAULT) + b1_ref[...]
  m = jnp.mean(h1, axis=0, keepdims=True)
  dev = h1 - m
  v = jnp.mean(dev * dev, axis=0, keepdims=True)
  h1 = jnp.maximum(g1_ref[...] * dev / jnp.sqrt(v + _EPS) + bb1_ref[...],
                   0.0)
  h2 = jnp.dot(h1, w2_ref[...],
               preferred_element_type=jnp.float32,
               precision=lax.Precision.DEFAULT) + b2_ref[...]
  m2 = jnp.mean(h2, axis=0, keepdims=True)
  dev2 = h2 - m2
  v2 = jnp.mean(dev2 * dev2, axis=0, keepdims=True)
  out_ref[...] = g2_ref[...] * dev2 / jnp.sqrt(v2 + _EPS) + bb2_ref[...]


def _head(x, p):
  b = x.shape[0]
  return pl.pallas_call(
      _head_body,
      out_shape=jax.ShapeDtypeStruct((b, p['fc2_W'].shape[1]), jnp.float32),
  )(x, p['fc1_W'], p['fc1_b'][None, :], p['fbn1_g'][None, :],
    p['fbn1_b'][None, :], p['fc2_W'], p['fc2_b'][None, :],
    p['fbn2_g'][None, :], p['fbn2_b'][None, :])


@jax.jit
def kernel(points, params):
  x = points[..., :3]
  b, n, d = x.shape
  x = jnp.pad(x, ((0, 0), (0, 0), (0, 8 - d)))
  hidden = [64, 64, 128, 128]
  prev = 3
  for i, hch in enumerate(hidden):
    up, um, s1p, s2p = _knn_conv(x, params['conv%d_W' % i],
                                 params['conv%d_b' % i][None, :], prev)
    x = _finalize(up, um, s1p, s2p,
                  params['cbn%d_g' % i][None, :],
                  params['cbn%d_b' % i][None, :],
                  params['bn%d_g' % i][None, :],
                  params['bn%d_b' % i][None, :])
    prev = hch
  return _head(x, params)
